# baseline (speedup 1.0000x reference)
import functools
import jax
import jax.numpy as jnp
from jax import lax
from jax.experimental import pallas as pl
from jax.experimental.pallas import tpu as pltpu
from jax.experimental.pallas import tpu_sc as plsc

_N = 4096
_NC = 2
_NS = 16
_NW = _NC * _NS
_ROWS_PER_W = _N // _NW
_BROWS = 8
_NCHUNK = _ROWS_PER_W // _BROWS


def _sc_outer_sum(x_hbm, out_hbm, x_v, buf_v, sem):
    wid = lax.axis_index("s") * _NC + lax.axis_index("c")
    base = wid * _ROWS_PER_W
    pltpu.sync_copy(x_hbm, x_v)

    def compute_chunk(c, slot):
        row0 = base + c * _BROWS
        rchunk = x_v[pl.ds(row0, 16)]
        rvecs = []
        for r in range(_BROWS):
            rvecs.append(jnp.full((16,), rchunk[r] + 1.0, jnp.float32))

        @plsc.parallel_loop(0, _N, step=16, unroll=8)
        def j_body(j):
            xc = x_v[pl.ds(j, 16)]
            for r in range(_BROWS):
                buf_v[slot, r, pl.ds(j, 16)] = xc + rvecs[r]

    def store_chunk(c, slot):
        row0 = base + c * _BROWS
        return pltpu.make_async_copy(
            buf_v.at[slot], out_hbm.at[pl.ds(row0, _BROWS)], sem)

    def outer_body(c2, _):
        for b in range(2):
            c = c2 * 2 + b

            @pl.when(c >= 2)
            def _wait():
                store_chunk(c, b).wait()

            compute_chunk(c, b)
            store_chunk(c, b).start()
        return 0

    lax.fori_loop(0, _NCHUNK // 2, outer_body, 0)
    store_chunk(_NCHUNK - 2, 0).wait()
    store_chunk(_NCHUNK - 1, 1).wait()


def kernel(x_leaves):
    n = x_leaves.shape[1]
    mesh = plsc.VectorSubcoreMesh(core_axis_name="c", subcore_axis_name="s")
    run = functools.partial(
        pl.kernel,
        mesh=mesh,
        out_type=jax.ShapeDtypeStruct((n, n), jnp.float32),
        scratch_types=[
            pltpu.VMEM((n,), jnp.float32),
            pltpu.VMEM((2, _BROWS, n), jnp.float32),
            pltpu.SemaphoreType.DMA,
        ],
    )(_sc_outer_sum)
    out = run(x_leaves.reshape(n))
    return out.reshape(1, n * n)


# speedup vs baseline: 1.2476x; 1.2476x over previous
import functools
import jax
import jax.numpy as jnp
from jax import lax
from jax.experimental import pallas as pl
from jax.experimental.pallas import tpu as pltpu
from jax.experimental.pallas import tpu_sc as plsc

_N = 4096
_NC = 2
_NS = 16
_NW = _NC * _NS
_ROWS_PER_W = _N // _NW
_BROWS = 8
_CHUNK = _BROWS * _N
_NCHUNK = _ROWS_PER_W // _BROWS


def _sc_outer_sum(x_hbm, out_hbm, x_v, buf_v, sem):
    wid = lax.axis_index("s") * _NC + lax.axis_index("c")
    base_row = wid * _ROWS_PER_W
    pltpu.sync_copy(x_hbm.at[0], x_v)

    def compute_chunk(c, slot):
        row0 = base_row + c * _BROWS
        rchunk = x_v[pl.ds(row0, 16)]
        rvecs = []
        for r in range(_BROWS):
            rvecs.append(jnp.full((16,), rchunk[r] + 1.0, jnp.float32))

        @plsc.parallel_loop(0, _N, step=16, unroll=4)
        def j_body(j):
            xc = x_v[pl.ds(j, 16)]
            for r in range(_BROWS):
                buf_v[slot, pl.ds(r * _N + j, 16)] = xc + rvecs[r]

    def store_chunk(c, slot):
        start = (base_row + c * _BROWS) * _N
        return pltpu.make_async_copy(
            buf_v.at[slot], out_hbm.at[0, pl.ds(start, _CHUNK)], sem)

    def outer_body(c2, _):
        for b in range(2):
            c = c2 * 2 + b

            @pl.when(c >= 2)
            def _wait():
                store_chunk(c, b).wait()

            compute_chunk(c, b)
            store_chunk(c, b).start()
        return 0

    lax.fori_loop(0, _NCHUNK // 2, outer_body, 0)
    store_chunk(_NCHUNK - 2, 0).wait()
    store_chunk(_NCHUNK - 1, 1).wait()


def kernel(x_leaves):
    n = x_leaves.shape[1]
    mesh = plsc.VectorSubcoreMesh(core_axis_name="c", subcore_axis_name="s")
    run = functools.partial(
        pl.kernel,
        mesh=mesh,
        out_type=jax.ShapeDtypeStruct((1, n * n), jnp.float32),
        scratch_types=[
            pltpu.VMEM((n,), jnp.float32),
            pltpu.VMEM((2, _CHUNK), jnp.float32),
            pltpu.SemaphoreType.DMA,
        ],
    )(_sc_outer_sum)
    return run(x_leaves)


# speedup vs baseline: 1.3070x; 1.0476x over previous
import functools
import jax
import jax.numpy as jnp
from jax import lax
from jax.experimental import pallas as pl
from jax.experimental.pallas import tpu as pltpu
from jax.experimental.pallas import tpu_sc as plsc

_N = 4096
_NC = 2
_NS = 16
_NW = _NC * _NS
_ROWS_PER_W = _N // _NW
_BROWS = 4
_CHUNK = _BROWS * _N
_NCHUNK = _ROWS_PER_W // _BROWS
_NBUF = 4


def _sc_outer_sum(x_hbm, out_hbm, x_v, buf_v, sem):
    wid = lax.axis_index("s") * _NC + lax.axis_index("c")
    base_row = wid * _ROWS_PER_W
    pltpu.sync_copy(x_hbm.at[0], x_v)

    def compute_chunk(g, b):
        rchunk = x_v[pl.ds(base_row + 16 * g, 16)]
        rvecs = []
        for r in range(_BROWS):
            rvecs.append(jnp.full((16,), rchunk[_BROWS * b + r] + 1.0,
                                  jnp.float32))

        @plsc.parallel_loop(0, _N, step=16, unroll=4)
        def j_body(j):
            xc = x_v[pl.ds(j, 16)]
            for r in range(_BROWS):
                buf_v[b, pl.ds(r * _N + j, 16)] = xc + rvecs[r]

    def store_chunk(c, b):
        start = (base_row + c * _BROWS) * _N
        return pltpu.make_async_copy(
            buf_v.at[b], out_hbm.at[0, pl.ds(start, _CHUNK)], sem)

    def outer_body(g, _):
        for b in range(_NBUF):
            c = g * _NBUF + b

            @pl.when(c >= _NBUF)
            def _wait():
                store_chunk(c, b).wait()

            compute_chunk(g, b)
            store_chunk(c, b).start()
        return 0

    lax.fori_loop(0, _NCHUNK // _NBUF, outer_body, 0)
    for b in range(_NBUF):
        store_chunk(_NCHUNK - _NBUF + b, b).wait()


def kernel(x_leaves):
    n = x_leaves.shape[1]
    mesh = plsc.VectorSubcoreMesh(core_axis_name="c", subcore_axis_name="s")
    run = functools.partial(
        pl.kernel,
        mesh=mesh,
        out_type=jax.ShapeDtypeStruct((1, n * n), jnp.float32),
        scratch_types=[
            pltpu.VMEM((n,), jnp.float32),
            pltpu.VMEM((_NBUF, _CHUNK), jnp.float32),
            pltpu.SemaphoreType.DMA,
        ],
    )(_sc_outer_sum)
    return run(x_leaves)
